# baseline (speedup 1.0000x reference)
import jax
import jax.numpy as jnp
from jax.experimental import pallas as pl
from jax.experimental.pallas import tpu as pltpu

_N = 10000
_BLOCK = 1000


def _cell_body(x_ref, h_ref, wx_ref, whzr_ref, whh_ref, wlin_ref, b_ref,
               blin_ref, out_ref, hnew_ref):
    x_b = x_ref[...]
    h_b = h_ref[...]
    g = jnp.dot(x_b, wx_ref[...], preferred_element_type=jnp.float32) + b_ref[...]
    zr = g[:, :64] + jnp.dot(h_b, whzr_ref[...], preferred_element_type=jnp.float32)
    z = jax.nn.sigmoid(zr[:, :32])
    r = jax.nn.sigmoid(zr[:, 32:])
    ht = jnp.tanh(g[:, 64:] + jnp.dot(h_b * r, whh_ref[...],
                                      preferred_element_type=jnp.float32))
    h_new = z * h_b + (1.0 - z) * ht
    hnew_ref[...] = h_new
    out_ref[...] = (jnp.dot(jnp.maximum(h_new, 0.0), wlin_ref[...],
                            preferred_element_type=jnp.float32) + blin_ref[...])


def kernel(x, edge_index, edge_weight, h, Wz, bz, Wr, br, Wh, bh, W_lin, b_lin):
    del edge_index, edge_weight
    d_in = x.shape[1]
    Wz_e = Wz[0, 0] + Wz[1, 0]
    Wr_e = Wr[0, 0] + Wr[1, 0]
    Wh_e = Wh[0, 0] + Wh[1, 0]
    wx = jnp.concatenate([Wz_e[:d_in], Wr_e[:d_in], Wh_e[:d_in]], axis=1)
    whzr = jnp.concatenate([Wz_e[d_in:], Wr_e[d_in:]], axis=1)
    whh = Wh_e[d_in:]
    b_all = jnp.concatenate([bz, br, bh])[None, :]
    blin = b_lin[None, :]

    grid = (_N // _BLOCK,)
    out, h_new = pl.pallas_call(
        _cell_body,
        grid=grid,
        in_specs=[
            pl.BlockSpec((_BLOCK, d_in), lambda i: (i, 0)),
            pl.BlockSpec((_BLOCK, h.shape[1]), lambda i: (i, 0)),
            pl.BlockSpec(wx.shape, lambda i: (0, 0)),
            pl.BlockSpec(whzr.shape, lambda i: (0, 0)),
            pl.BlockSpec(whh.shape, lambda i: (0, 0)),
            pl.BlockSpec(W_lin.shape, lambda i: (0, 0)),
            pl.BlockSpec(b_all.shape, lambda i: (0, 0)),
            pl.BlockSpec(blin.shape, lambda i: (0, 0)),
        ],
        out_specs=[
            pl.BlockSpec((_BLOCK, W_lin.shape[1]), lambda i: (i, 0)),
            pl.BlockSpec((_BLOCK, h.shape[1]), lambda i: (i, 0)),
        ],
        out_shape=[
            jax.ShapeDtypeStruct((_N, W_lin.shape[1]), jnp.float32),
            jax.ShapeDtypeStruct((_N, h.shape[1]), jnp.float32),
        ],
        compiler_params=pltpu.CompilerParams(
            dimension_semantics=("parallel",),
        ),
    )(x, h, wx, whzr, whh, W_lin, b_all, blin)
    return (out, h_new)


# speedup vs baseline: 1.1396x; 1.1396x over previous
import jax
import jax.numpy as jnp
from jax.experimental import pallas as pl
from jax.experimental.pallas import tpu as pltpu

_N = 10000
_BLOCK = 2000


def _cell_body(x_ref, h_ref, wz_ref, wr_ref, wh_ref, bz_ref, br_ref, bh_ref,
               wlin_ref, blin_ref, out_ref, hnew_ref):
    d_in = x_ref.shape[1]
    wz = wz_ref[0, 0] + wz_ref[1, 0]
    wr = wr_ref[0, 0] + wr_ref[1, 0]
    wh = wh_ref[0, 0] + wh_ref[1, 0]
    x_b = x_ref[...]
    h_b = h_ref[...]
    z = jax.nn.sigmoid(
        jnp.dot(x_b, wz[:d_in], preferred_element_type=jnp.float32)
        + jnp.dot(h_b, wz[d_in:], preferred_element_type=jnp.float32)
        + bz_ref[...])
    r = jax.nn.sigmoid(
        jnp.dot(x_b, wr[:d_in], preferred_element_type=jnp.float32)
        + jnp.dot(h_b, wr[d_in:], preferred_element_type=jnp.float32)
        + br_ref[...])
    ht = jnp.tanh(
        jnp.dot(x_b, wh[:d_in], preferred_element_type=jnp.float32)
        + jnp.dot(h_b * r, wh[d_in:], preferred_element_type=jnp.float32)
        + bh_ref[...])
    h_new = z * h_b + (1.0 - z) * ht
    hnew_ref[...] = h_new
    out_ref[...] = (jnp.dot(jnp.maximum(h_new, 0.0), wlin_ref[...],
                            preferred_element_type=jnp.float32) + blin_ref[...])


def kernel(x, edge_index, edge_weight, h, Wz, bz, Wr, br, Wh, bh, W_lin, b_lin):
    del edge_index, edge_weight
    d_hid = h.shape[1]
    bz2, br2, bh2, blin2 = bz[None], br[None], bh[None], b_lin[None]

    grid = (_N // _BLOCK,)
    full = lambda a: pl.BlockSpec(a.shape, lambda i: (0,) * a.ndim)
    out, h_new = pl.pallas_call(
        _cell_body,
        grid=grid,
        in_specs=[
            pl.BlockSpec((_BLOCK, x.shape[1]), lambda i: (i, 0)),
            pl.BlockSpec((_BLOCK, d_hid), lambda i: (i, 0)),
            full(Wz), full(Wr), full(Wh),
            full(bz2), full(br2), full(bh2),
            full(W_lin), full(blin2),
        ],
        out_specs=[
            pl.BlockSpec((_BLOCK, W_lin.shape[1]), lambda i: (i, 0)),
            pl.BlockSpec((_BLOCK, d_hid), lambda i: (i, 0)),
        ],
        out_shape=[
            jax.ShapeDtypeStruct((_N, W_lin.shape[1]), jnp.float32),
            jax.ShapeDtypeStruct((_N, d_hid), jnp.float32),
        ],
        compiler_params=pltpu.CompilerParams(
            dimension_semantics=("parallel",),
        ),
    )(x, h, Wz, Wr, Wh, bz2, br2, bh2, W_lin, blin2)
    return (out, h_new)
